# baseline (speedup 1.0000x reference)
import functools
import math

import jax
import jax.numpy as jnp
from jax import lax
from jax.experimental import pallas as pl
from jax.experimental.pallas import tpu as pltpu
from jax.experimental.pallas import tpu_sc as plsc

D_MODEL = 64
SEQ_LEN = 200
BATCH = 4096
MAX_LEN = 350

_NC = 2
_NS = 16
_NW = _NC * _NS
_B_TOTAL = BATCH * SEQ_LEN
_B_PER_W = _B_TOTAL // _NW
_C = 40
_P = SEQ_LEN // _C
_G = _B_PER_W // _C
_K = _P
_DPAD = 128
_GR = _G // _K
_GROUPS = D_MODEL // 16
_SEQ_PER_W = _B_PER_W // SEQ_LEN


def _sin_pos_encoding():
    position = jnp.arange(0, MAX_LEN, dtype=jnp.float32)[:, None]
    divisor = jnp.exp(
        jnp.arange(0, D_MODEL, 2, dtype=jnp.float32)
        * -(math.log(10000.0) / D_MODEL)
    )
    pe = jnp.zeros((MAX_LEN, D_MODEL), dtype=jnp.float32)
    pe = pe.at[:, 0::2].set(jnp.sin(position * divisor))
    pe = pe.at[:, 1::2].set(jnp.cos(position * divisor))
    return pe[:SEQ_LEN]


@functools.partial(
    pl.kernel,
    out_type=jax.ShapeDtypeStruct((BATCH, SEQ_LEN, D_MODEL), jnp.float32),
    mesh=plsc.VectorSubcoreMesh(core_axis_name="c", subcore_axis_name="s"),
    scratch_types=[
        pltpu.VMEM((_B_PER_W,), jnp.int32),
        pltpu.VMEM((_K, _C, _DPAD), jnp.float32),
        pltpu.VMEM((_K, _C, D_MODEL), jnp.float32),
        pltpu.VMEM((SEQ_LEN, D_MODEL), jnp.float32),
        pltpu.SemaphoreType.DMA((_K,)),
        pltpu.SemaphoreType.DMA((_K,)),
    ],
    compiler_params=pltpu.CompilerParams(use_tc_tiling_on_sc=True),
)
def _sc_embed(table_hbm, x_hbm, pe_hbm, out_hbm, idx_v, rows_r, o_r, pe_v, sg, sw):
    cid = lax.axis_index("c")
    sid = lax.axis_index("s")
    wid = sid * _NC + cid
    base = wid * _B_PER_W
    scale = float(math.sqrt(D_MODEL))

    pltpu.sync_copy(pe_hbm, pe_v)
    pltpu.sync_copy(x_hbm.at[pl.ds(base, _B_PER_W)], idx_v)

    def gather(g, b):
        return pltpu.make_async_copy(
            table_hbm.at[idx_v.at[pl.ds(g * _C, _C)]], rows_r.at[b], sg.at[b])

    def writeback(g, b, part):
        seq = wid * _SEQ_PER_W + g // _P
        return pltpu.make_async_copy(
            o_r.at[b], out_hbm.at[seq, pl.ds(part * _C, _C)], sw.at[b])

    for b in range(_K):
        gather(b, b).start()

    def group_body(t, carry):
        for b in range(_K):
            g = t * _K + b
            part = b % _P
            gather(g, b).wait()

            ph = part * _C

            for s in range(_C):
                for gi in range(_GROUPS):
                    sl = pl.ds(gi * 16, 16)
                    o_r[b, s, sl] = rows_r[b, s, sl] * scale + pe_v[ph + s, sl]
            writeback(g, b, part).start()

        @pl.when(t + 1 < _GR)
        def _fire_next():
            for b in range(_K):
                g = t * _K + b
                writeback(g, b, b % _P).wait()
                gather(g + _K, b).start()

        return carry

    lax.fori_loop(0, _GR, group_body, 0)

    for b in range(_K):
        writeback((_GR - 1) * _K + b, b, b % _P).wait()


def kernel(x, emb_table):
    pe = _sin_pos_encoding()
    table128 = jnp.pad(emb_table, ((0, 0), (0, _DPAD - D_MODEL)))
    return _sc_embed(table128, x.reshape(-1), pe)


# speedup vs baseline: 1.2891x; 1.2891x over previous
import functools
import math

import jax
import jax.numpy as jnp
from jax import lax
from jax.experimental import pallas as pl
from jax.experimental.pallas import tpu as pltpu
from jax.experimental.pallas import tpu_sc as plsc

D_MODEL = 64
SEQ_LEN = 200
BATCH = 4096
MAX_LEN = 350

_NC = 2
_NS = 16
_NW = _NC * _NS
_B_TOTAL = BATCH * SEQ_LEN
_B_PER_W = _B_TOTAL // _NW
_C = 40
_P = SEQ_LEN // _C
_G = _B_PER_W // _C
_K = _P
_DPAD = 128
_GR = _G // _K
_GROUPS = D_MODEL // 16
_SEQ_PER_W = _B_PER_W // SEQ_LEN


def _sin_pos_encoding():
    position = jnp.arange(0, MAX_LEN, dtype=jnp.float32)[:, None]
    divisor = jnp.exp(
        jnp.arange(0, D_MODEL, 2, dtype=jnp.float32)
        * -(math.log(10000.0) / D_MODEL)
    )
    pe = jnp.zeros((MAX_LEN, D_MODEL), dtype=jnp.float32)
    pe = pe.at[:, 0::2].set(jnp.sin(position * divisor))
    pe = pe.at[:, 1::2].set(jnp.cos(position * divisor))
    return pe[:SEQ_LEN]


@functools.partial(
    pl.kernel,
    out_type=jax.ShapeDtypeStruct((BATCH, SEQ_LEN, D_MODEL), jnp.float32),
    mesh=plsc.VectorSubcoreMesh(core_axis_name="c", subcore_axis_name="s"),
    scratch_types=[
        pltpu.VMEM((_B_PER_W,), jnp.int32),
        pltpu.VMEM((_K, _C, _DPAD), jnp.float32),
        pltpu.VMEM((_K, _C, D_MODEL), jnp.float32),
        pltpu.VMEM((SEQ_LEN, D_MODEL), jnp.float32),
        pltpu.SemaphoreType.DMA((_K,)),
        pltpu.SemaphoreType.DMA((_K,)),
    ],
    compiler_params=pltpu.CompilerParams(use_tc_tiling_on_sc=True),
)
def _sc_embed(table_hbm, x_hbm, pe_hbm, out_hbm, idx_v, rows_r, o_r, pe_v, sg, sw):
    cid = lax.axis_index("c")
    sid = lax.axis_index("s")
    wid = sid * _NC + cid
    base = wid * _B_PER_W
    scale = float(math.sqrt(D_MODEL))

    pltpu.sync_copy(pe_hbm, pe_v)
    pltpu.sync_copy(x_hbm.at[pl.ds(base, _B_PER_W)], idx_v)

    def gather(g, b):
        return pltpu.make_async_copy(
            table_hbm.at[idx_v.at[pl.ds(g * _C, _C)]], rows_r.at[b], sg.at[b])

    def writeback(g, b, part):
        seq = wid * _SEQ_PER_W + g // _P
        return pltpu.make_async_copy(
            o_r.at[b], out_hbm.at[seq, pl.ds(part * _C, _C)], sw.at[b])

    for b in range(_K):
        gather(b, b).start()

    def group_body(t, carry):
        for b in range(_K):
            g = t * _K + b
            part = b % _P
            gather(g, b).wait()

            @pl.when(t > 0)
            def _drain(g=g, b=b, part=part):
                writeback(g - _K, b, part).wait()

            ph = part * _C

            def row_body(s, c2, b=b, ph=ph):
                for gi in range(_GROUPS):
                    sl = pl.ds(gi * 16, 16)
                    o_r[b, s, sl] = rows_r[b, s, sl] * scale + pe_v[ph + s, sl]
                return c2

            lax.fori_loop(0, _C, row_body, 0)
            writeback(g, b, part).start()

            @pl.when(t + 1 < _GR)
            def _refire(g=g, b=b):
                gather(g + _K, b).start()

        return carry

    lax.fori_loop(0, _GR, group_body, 0)

    for b in range(_K):
        writeback((_GR - 1) * _K + b, b, b % _P).wait()


def kernel(x, emb_table):
    pe = _sin_pos_encoding()
    table128 = jnp.pad(emb_table, ((0, 0), (0, _DPAD - D_MODEL)))
    return _sc_embed(table128, x.reshape(-1), pe)
